# baseline (speedup 1.0000x reference)
import functools

import jax
import jax.numpy as jnp
from jax import lax
from jax.experimental import pallas as pl
from jax.experimental.pallas import tpu as pltpu
from jax.experimental.pallas import tpu_sc as plsc

EMBED = 64
BATCH = 16384

_NC, _NS = 2, 16
_NW = _NC * _NS
_BPW = BATCH // _NW


@functools.cache
def _make_gather():
    mesh = plsc.VectorSubcoreMesh(
        core_axis_name="c", subcore_axis_name="s", num_cores=_NC)

    @functools.partial(
        pl.kernel,
        mesh=mesh,
        compiler_params=pltpu.CompilerParams(use_tc_tiling_on_sc=False),
        out_type=[
            jax.ShapeDtypeStruct((EMBED, BATCH), jnp.float32),
            jax.ShapeDtypeStruct((EMBED, BATCH), jnp.float32),
        ],
        scratch_types=[
            pltpu.VMEM((_BPW,), jnp.int32),
            pltpu.VMEM((EMBED, _BPW), jnp.float32),
            pltpu.VMEM((_BPW,), jnp.int32),
            pltpu.VMEM((EMBED, _BPW), jnp.float32),
            pltpu.SemaphoreType.DMA,
            pltpu.SemaphoreType.DMA,
        ],
    )
    def gather_kernel(user_hbm, movie_hbm, utabT_hbm, mtabT_hbm, uout_hbm,
                      mout_hbm, uidx_v, urows_v, midx_v, mrows_v, usem, msem):
        wid = lax.axis_index("s") * _NC + lax.axis_index("c")
        base = wid * _BPW
        pltpu.sync_copy(user_hbm.at[pl.ds(base, _BPW)], uidx_v)
        pltpu.sync_copy(movie_hbm.at[pl.ds(base, _BPW)], midx_v)

        def fire(j, _):
            pltpu.async_copy(utabT_hbm.at[j].at[uidx_v], urows_v.at[j], usem)
            pltpu.async_copy(mtabT_hbm.at[j].at[midx_v], mrows_v.at[j], msem)
            return 0

        lax.fori_loop(0, EMBED, fire, 0)
        pltpu.make_async_copy(
            utabT_hbm.at[:, pl.ds(0, _BPW)], urows_v, usem).wait()
        pltpu.make_async_copy(
            mtabT_hbm.at[:, pl.ds(0, _BPW)], mrows_v, msem).wait()
        pltpu.sync_copy(urows_v, uout_hbm.at[:, pl.ds(base, _BPW)])
        pltpu.sync_copy(mrows_v, mout_hbm.at[:, pl.ds(base, _BPW)])

    return gather_kernel


_BLK = 2048
_CONTRACT0 = (((0,), (0,)), ((), ()))


def _mlp_body(u_ref, m_ref, w1u_ref, w1m_ref, b1_ref, w2_ref, b2_ref, w3_ref,
              b3_ref, w4_ref, b4_ref, w5_ref, b5_ref, out_ref):
    x = (lax.dot_general(u_ref[...], w1u_ref[...], _CONTRACT0,
                         preferred_element_type=jnp.float32)
         + lax.dot_general(m_ref[...], w1m_ref[...], _CONTRACT0,
                           preferred_element_type=jnp.float32)
         + b1_ref[...])
    x = jnp.maximum(x, 0.0)
    x = jnp.maximum(x @ w2_ref[...] + b2_ref[...], 0.0)
    x = jnp.maximum(x @ w3_ref[...] + b3_ref[...], 0.0)
    x = jnp.maximum(x @ w4_ref[...] + b4_ref[...], 0.0)
    out_ref[...] = x @ w5_ref[...] + b5_ref[...]


def _mlp(uT, mT, W1u, W1m, b1, W2, b2, W3, b3, W4, b4, W5, b5):
    grid = (BATCH // _BLK,)
    col_spec = pl.BlockSpec((EMBED, _BLK), lambda i: (0, i))
    full = lambda a: pl.BlockSpec(a.shape, lambda i: (0,) * a.ndim)
    in_specs = [col_spec, col_spec] + [
        full(a) for a in (W1u, W1m, b1, W2, b2, W3, b3, W4, b4, W5, b5)
    ]
    return pl.pallas_call(
        _mlp_body,
        grid=grid,
        in_specs=in_specs,
        out_specs=pl.BlockSpec((_BLK, 1), lambda i: (i, 0)),
        out_shape=jax.ShapeDtypeStruct((BATCH, 1), jnp.float32),
        compiler_params=pltpu.CompilerParams(
            dimension_semantics=("arbitrary",),
        ),
    )(uT, mT, W1u, W1m, b1, W2, b2, W3, b3, W4, b4, W5, b5)


def kernel(user, movie, user_table, movie_table, W1, b1, W2, b2, W3, b3, W4,
           b4, W5, b5):
    uT, mT = _make_gather()(user.astype(jnp.int32), movie.astype(jnp.int32),
                            user_table.T, movie_table.T)
    return _mlp(uT, mT, W1[:EMBED], W1[EMBED:], b1.reshape(1, -1),
                W2, b2.reshape(1, -1), W3, b3.reshape(1, -1),
                W4, b4.reshape(1, -1), W5, b5.reshape(1, -1))


# speedup vs baseline: 8.7873x; 8.7873x over previous
import functools

import jax
import jax.numpy as jnp
from jax import lax
from jax.experimental import pallas as pl
from jax.experimental.pallas import tpu as pltpu
from jax.experimental.pallas import tpu_sc as plsc

EMBED = 64
BATCH = 16384
PAIRED = 2 * EMBED

_NC, _NS = 2, 16
_NW = _NC * _NS
_BPW = BATCH // _NW
_L = 16


@functools.cache
def _make_gather():
    mesh = plsc.VectorSubcoreMesh(
        core_axis_name="c", subcore_axis_name="s", num_cores=_NC)

    @functools.partial(
        pl.kernel,
        mesh=mesh,
        compiler_params=pltpu.CompilerParams(use_tc_tiling_on_sc=False),
        out_type=[
            jax.ShapeDtypeStruct((BATCH, PAIRED), jnp.float32),
            jax.ShapeDtypeStruct((BATCH, PAIRED), jnp.float32),
        ],
        scratch_types=[
            pltpu.VMEM((_BPW,), jnp.int32),
            pltpu.VMEM((_BPW,), jnp.int32),
            pltpu.VMEM((_BPW, PAIRED), jnp.float32),
            pltpu.SemaphoreType.DMA,
        ],
    )
    def gather_kernel(user_hbm, movie_hbm, utab_hbm, mtab_hbm, uout_hbm,
                      mout_hbm, idx_v, row_v, gbuf_v, sem):
        wid = lax.axis_index("s") * _NC + lax.axis_index("c")
        base = wid * _BPW

        def one_table(idx_hbm, tab_hbm, out_hbm):
            pltpu.sync_copy(idx_hbm.at[pl.ds(base, _BPW)], idx_v)

            def halve(k, _):
                v = idx_v[pl.ds(k * _L, _L)]
                row_v[pl.ds(k * _L, _L)] = lax.shift_right_logical(v, 1)
                return 0

            lax.fori_loop(0, _BPW // _L, halve, 0)
            pltpu.async_copy(tab_hbm.at[row_v], gbuf_v, sem).wait()
            pltpu.sync_copy(gbuf_v, out_hbm.at[pl.ds(base, _BPW)])

        one_table(user_hbm, utab_hbm, uout_hbm)
        one_table(movie_hbm, mtab_hbm, mout_hbm)

    return gather_kernel


_BLK = 2048


def _mlp_body(pu_ref, pm_ref, eu_ref, em_ref, w1u_ref, w1m_ref, b1_ref,
              w2_ref, b2_ref, w3_ref, b3_ref, w4_ref, b4_ref, w5_ref, b5_ref,
              out_ref):
    eu = eu_ref[...]
    em = em_ref[...]
    u = pu_ref[:, :EMBED] * (1.0 - eu) + pu_ref[:, EMBED:] * eu
    m = pm_ref[:, :EMBED] * (1.0 - em) + pm_ref[:, EMBED:] * em
    x = u @ w1u_ref[...] + m @ w1m_ref[...] + b1_ref[...]
    x = jnp.maximum(x, 0.0)
    x = jnp.maximum(x @ w2_ref[...] + b2_ref[...], 0.0)
    x = jnp.maximum(x @ w3_ref[...] + b3_ref[...], 0.0)
    x = jnp.maximum(x @ w4_ref[...] + b4_ref[...], 0.0)
    out_ref[...] = x @ w5_ref[...] + b5_ref[...]


def _mlp(pu, pm, eu, em, W1u, W1m, b1, W2, b2, W3, b3, W4, b4, W5, b5):
    grid = (BATCH // _BLK,)
    row_spec = lambda d: pl.BlockSpec((_BLK, d), lambda i: (i, 0))
    full = lambda a: pl.BlockSpec(a.shape, lambda i: (0,) * a.ndim)
    in_specs = [row_spec(PAIRED), row_spec(PAIRED), row_spec(1), row_spec(1)]
    in_specs += [
        full(a) for a in (W1u, W1m, b1, W2, b2, W3, b3, W4, b4, W5, b5)
    ]
    return pl.pallas_call(
        _mlp_body,
        grid=grid,
        in_specs=in_specs,
        out_specs=pl.BlockSpec((_BLK, 1), lambda i: (i, 0)),
        out_shape=jax.ShapeDtypeStruct((BATCH, 1), jnp.float32),
        compiler_params=pltpu.CompilerParams(
            dimension_semantics=("arbitrary",),
        ),
    )(pu, pm, eu, em, W1u, W1m, b1, W2, b2, W3, b3, W4, b4, W5, b5)


def kernel(user, movie, user_table, movie_table, W1, b1, W2, b2, W3, b3, W4,
           b4, W5, b5):
    user = user.astype(jnp.int32)
    movie = movie.astype(jnp.int32)
    u2 = user_table.reshape(-1, PAIRED)
    m2 = movie_table.reshape(-1, PAIRED)
    pu, pm = _make_gather()(user, movie, u2, m2)
    eu = (user & 1).astype(jnp.float32).reshape(-1, 1)
    em = (movie & 1).astype(jnp.float32).reshape(-1, 1)
    return _mlp(pu, pm, eu, em, W1[:EMBED], W1[EMBED:], b1.reshape(1, -1),
                W2, b2.reshape(1, -1), W3, b3.reshape(1, -1),
                W4, b4.reshape(1, -1), W5, b5.reshape(1, -1))


# speedup vs baseline: 8.8031x; 1.0018x over previous
import functools

import jax
import jax.numpy as jnp
from jax import lax
from jax.experimental import pallas as pl
from jax.experimental.pallas import tpu as pltpu
from jax.experimental.pallas import tpu_sc as plsc

EMBED = 64
BATCH = 16384
PAIRED = 2 * EMBED

_NC, _NS = 2, 16
_NW = _NC * _NS
_BPW = BATCH // _NW
_L = 16


@functools.cache
def _make_gather():
    mesh = plsc.VectorSubcoreMesh(
        core_axis_name="c", subcore_axis_name="s", num_cores=_NC)

    @functools.partial(
        pl.kernel,
        mesh=mesh,
        compiler_params=pltpu.CompilerParams(use_tc_tiling_on_sc=True),
        out_type=[
            jax.ShapeDtypeStruct((BATCH, PAIRED), jnp.float32),
            jax.ShapeDtypeStruct((BATCH, PAIRED), jnp.float32),
        ],
        scratch_types=[
            pltpu.VMEM((_BPW,), jnp.int32),
            pltpu.VMEM((_BPW,), jnp.int32),
            pltpu.VMEM((_BPW, PAIRED), jnp.float32),
            pltpu.SemaphoreType.DMA,
        ],
    )
    def gather_kernel(user_hbm, movie_hbm, utab_hbm, mtab_hbm, uout_hbm,
                      mout_hbm, idx_v, row_v, gbuf_v, sem):
        wid = lax.axis_index("s") * _NC + lax.axis_index("c")
        base = wid * _BPW

        def one_table(idx_hbm, tab_hbm, out_hbm):
            pltpu.sync_copy(idx_hbm.at[pl.ds(base, _BPW)], idx_v)

            def halve(k, _):
                v = idx_v[pl.ds(k * _L, _L)]
                row_v[pl.ds(k * _L, _L)] = lax.shift_right_logical(v, 1)
                return 0

            lax.fori_loop(0, _BPW // _L, halve, 0)
            pltpu.async_copy(tab_hbm.at[row_v], gbuf_v, sem).wait()
            pltpu.sync_copy(gbuf_v, out_hbm.at[pl.ds(base, _BPW)])

        one_table(user_hbm, utab_hbm, uout_hbm)
        one_table(movie_hbm, mtab_hbm, mout_hbm)

    return gather_kernel


_BLK = 2048


def _mlp_body(pu_ref, pm_ref, eu_ref, em_ref, w1u_ref, w1m_ref, b1_ref,
              w2_ref, b2_ref, w3_ref, b3_ref, w4_ref, b4_ref, w5_ref, b5_ref,
              out_ref):
    eu = eu_ref[...]
    em = em_ref[...]
    u = pu_ref[:, :EMBED] * (1.0 - eu) + pu_ref[:, EMBED:] * eu
    m = pm_ref[:, :EMBED] * (1.0 - em) + pm_ref[:, EMBED:] * em
    x = u @ w1u_ref[...] + m @ w1m_ref[...] + b1_ref[...]
    x = jnp.maximum(x, 0.0)
    x = jnp.maximum(x @ w2_ref[...] + b2_ref[...], 0.0)
    x = jnp.maximum(x @ w3_ref[...] + b3_ref[...], 0.0)
    x = jnp.maximum(x @ w4_ref[...] + b4_ref[...], 0.0)
    out_ref[...] = x @ w5_ref[...] + b5_ref[...]


def _mlp(pu, pm, eu, em, W1u, W1m, b1, W2, b2, W3, b3, W4, b4, W5, b5):
    grid = (BATCH // _BLK,)
    row_spec = lambda d: pl.BlockSpec((_BLK, d), lambda i: (i, 0))
    full = lambda a: pl.BlockSpec(a.shape, lambda i: (0,) * a.ndim)
    in_specs = [row_spec(PAIRED), row_spec(PAIRED), row_spec(1), row_spec(1)]
    in_specs += [
        full(a) for a in (W1u, W1m, b1, W2, b2, W3, b3, W4, b4, W5, b5)
    ]
    return pl.pallas_call(
        _mlp_body,
        grid=grid,
        in_specs=in_specs,
        out_specs=pl.BlockSpec((_BLK, 1), lambda i: (i, 0)),
        out_shape=jax.ShapeDtypeStruct((BATCH, 1), jnp.float32),
        compiler_params=pltpu.CompilerParams(
            dimension_semantics=("arbitrary",),
        ),
    )(pu, pm, eu, em, W1u, W1m, b1, W2, b2, W3, b3, W4, b4, W5, b5)


def kernel(user, movie, user_table, movie_table, W1, b1, W2, b2, W3, b3, W4,
           b4, W5, b5):
    user = user.astype(jnp.int32)
    movie = movie.astype(jnp.int32)
    u2 = user_table.reshape(-1, PAIRED)
    m2 = movie_table.reshape(-1, PAIRED)
    pu, pm = _make_gather()(user, movie, u2, m2)
    eu = (user & 1).astype(jnp.float32).reshape(-1, 1)
    em = (movie & 1).astype(jnp.float32).reshape(-1, 1)
    return _mlp(pu, pm, eu, em, W1[:EMBED], W1[EMBED:], b1.reshape(1, -1),
                W2, b2.reshape(1, -1), W3, b3.reshape(1, -1),
                W4, b4.reshape(1, -1), W5, b5.reshape(1, -1))


# speedup vs baseline: 8.8683x; 1.0074x over previous
import functools

import jax
import jax.numpy as jnp
from jax import lax
from jax.experimental import pallas as pl
from jax.experimental.pallas import tpu as pltpu
from jax.experimental.pallas import tpu_sc as plsc

EMBED = 64
BATCH = 16384

_NC, _NS = 2, 16
_NW = _NC * _NS
_BPW = BATCH // _NW


@functools.cache
def _make_gather():
    mesh = plsc.VectorSubcoreMesh(
        core_axis_name="c", subcore_axis_name="s", num_cores=_NC)

    @functools.partial(
        pl.kernel,
        mesh=mesh,
        compiler_params=pltpu.CompilerParams(use_tc_tiling_on_sc=False),
        out_type=[
            jax.ShapeDtypeStruct((BATCH, EMBED), jnp.float32),
            jax.ShapeDtypeStruct((BATCH, EMBED), jnp.float32),
        ],
        scratch_types=[
            pltpu.VMEM((_BPW,), jnp.int32),
            pltpu.VMEM((_BPW, EMBED), jnp.float32),
            pltpu.VMEM((_BPW,), jnp.int32),
            pltpu.VMEM((_BPW, EMBED), jnp.float32),
            pltpu.SemaphoreType.DMA,
            pltpu.SemaphoreType.DMA,
        ],
    )
    def gather_kernel(user_hbm, movie_hbm, utab_hbm, mtab_hbm, uout_hbm,
                      mout_hbm, uidx_v, urows_v, midx_v, mrows_v, usem, msem):
        wid = lax.axis_index("s") * _NC + lax.axis_index("c")
        base = wid * _BPW
        pltpu.sync_copy(user_hbm.at[pl.ds(base, _BPW)], uidx_v)
        pltpu.sync_copy(movie_hbm.at[pl.ds(base, _BPW)], midx_v)
        ucp = pltpu.async_copy(utab_hbm.at[uidx_v], urows_v, usem)
        mcp = pltpu.async_copy(mtab_hbm.at[midx_v], mrows_v, msem)
        ucp.wait()
        pltpu.sync_copy(urows_v, uout_hbm.at[pl.ds(base, _BPW)])
        mcp.wait()
        pltpu.sync_copy(mrows_v, mout_hbm.at[pl.ds(base, _BPW)])

    return gather_kernel


_BLK = 2048


def _mlp_body(u_ref, m_ref, w1u_ref, w1m_ref, b1_ref, w2_ref, b2_ref, w3_ref,
              b3_ref, w4_ref, b4_ref, w5_ref, b5_ref, out_ref):
    x = u_ref[...] @ w1u_ref[...] + m_ref[...] @ w1m_ref[...] + b1_ref[...]
    x = jnp.maximum(x, 0.0)
    x = jnp.maximum(x @ w2_ref[...] + b2_ref[...], 0.0)
    x = jnp.maximum(x @ w3_ref[...] + b3_ref[...], 0.0)
    x = jnp.maximum(x @ w4_ref[...] + b4_ref[...], 0.0)
    out_ref[...] = x @ w5_ref[...] + b5_ref[...]


def _mlp(u, m, W1u, W1m, b1, W2, b2, W3, b3, W4, b4, W5, b5):
    grid = (BATCH // _BLK,)
    row_spec = pl.BlockSpec((_BLK, EMBED), lambda i: (i, 0))
    full = lambda a: pl.BlockSpec(a.shape, lambda i: (0,) * a.ndim)
    in_specs = [row_spec, row_spec] + [
        full(a) for a in (W1u, W1m, b1, W2, b2, W3, b3, W4, b4, W5, b5)
    ]
    return pl.pallas_call(
        _mlp_body,
        grid=grid,
        in_specs=in_specs,
        out_specs=pl.BlockSpec((_BLK, 1), lambda i: (i, 0)),
        out_shape=jax.ShapeDtypeStruct((BATCH, 1), jnp.float32),
        compiler_params=pltpu.CompilerParams(
            dimension_semantics=("parallel",),
        ),
    )(u, m, W1u, W1m, b1, W2, b2, W3, b3, W4, b4, W5, b5)


def kernel(user, movie, user_table, movie_table, W1, b1, W2, b2, W3, b3, W4,
           b4, W5, b5):
    u, m = _make_gather()(user.astype(jnp.int32), movie.astype(jnp.int32),
                          user_table, movie_table)
    return _mlp(u, m, W1[:EMBED], W1[EMBED:], b1.reshape(1, -1),
                W2, b2.reshape(1, -1), W3, b3.reshape(1, -1),
                W4, b4.reshape(1, -1), W5, b5.reshape(1, -1))
